# baseline (speedup 1.0000x reference)
import jax
import jax.numpy as jnp
from jax.experimental import pallas as pl

N = 10000
G = 64


def _identity_body(x_ref, o_ref):
    o_ref[...] = x_ref[...]


def _gcn_conv(x, edge_index, W, b):
    h = x @ W
    src = edge_index[0]
    dst = edge_index[1]
    loop = jnp.arange(N, dtype=src.dtype)
    src = jnp.concatenate([src, loop])
    dst = jnp.concatenate([dst, loop])
    ones = jnp.ones(src.shape[0], dtype=h.dtype)
    deg = jax.ops.segment_sum(ones, dst, num_segments=N)
    dinv = jnp.where(deg > 0, 1.0 / jnp.sqrt(deg), 0.0)
    norm = dinv[src] * dinv[dst]
    msg = h[src] * norm[:, None]
    out = jax.ops.segment_sum(msg, dst, num_segments=N)
    return out + b


def kernel(x, edge_index, batch, W1, b1, W2, b2, Wfc, bfc):
    h = jax.nn.relu(_gcn_conv(x, edge_index, W1, b1))
    h = jax.nn.relu(_gcn_conv(h, edge_index, W2, b2))
    h = h @ Wfc + bfc
    sums = jax.ops.segment_sum(h, batch, num_segments=G)
    counts = jax.ops.segment_sum(jnp.ones((N, 1), dtype=h.dtype), batch, num_segments=G)
    out = sums / jnp.maximum(counts, 1.0)
    return pl.pallas_call(
        _identity_body,
        out_shape=jax.ShapeDtypeStruct(out.shape, out.dtype),
    )(out)


# speedup vs baseline: 10.5195x; 10.5195x over previous
import functools

import jax
import jax.numpy as jnp
from jax import lax
from jax.experimental import pallas as pl
from jax.experimental.pallas import tpu as pltpu
from jax.experimental.pallas import tpu_sc as plsc

N = 10000
E = 320000
G = 64
D_IN = 128
D_H = 256
HALF = 128

NC = 2
NS = 16
LANES = 16

CHUNK = 128
IDX_STAGE = 40
E_PAD = 327680
NCHUNK = E_PAD // CHUNK
ROWS_PER_TILE = 624
TAIL_ROWS = N - NS * ROWS_PER_TILE
ACC_ROWS = N + LANES

RB = 2000
NB = N // RB


def _sc_degree_body(edst_hbm, ones_hbm, init_hbm, out_hbm, dst_v, ones_v,
                    deg_sh, sem):
    c = lax.axis_index("c")
    s = lax.axis_index("s")
    rbase = s * ROWS_PER_TILE
    pltpu.sync_copy(init_hbm.at[pl.ds(c * ACC_ROWS + rbase, ROWS_PER_TILE)],
                    deg_sh.at[pl.ds(rbase, ROWS_PER_TILE)])
    @pl.when(s == NS - 1)
    def _():
        pltpu.sync_copy(
            init_hbm.at[pl.ds(c * ACC_ROWS + NS * ROWS_PER_TILE,
                              TAIL_ROWS + LANES)],
            deg_sh.at[pl.ds(NS * ROWS_PER_TILE, TAIL_ROWS + LANES)])
    pltpu.sync_copy(ones_hbm, ones_v)

    per_tile = NCHUNK // (NC * NS)
    cbase = (c * NS + s) * per_tile
    pltpu.sync_copy(edst_hbm.at[pl.ds(cbase, per_tile)], dst_v)
    plsc.subcore_barrier()

    def chunk(k, carry):
        pltpu.sync_copy(ones_v, deg_sh.at[dst_v.at[k]], add=True)
        return carry
    lax.fori_loop(0, per_tile, chunk, 0)

    plsc.subcore_barrier()
    pltpu.sync_copy(deg_sh.at[pl.ds(rbase, ROWS_PER_TILE)],
                    out_hbm.at[pl.ds(c * N + rbase, ROWS_PER_TILE)])
    @pl.when(s == NS - 1)
    def _():
        pltpu.sync_copy(
            deg_sh.at[pl.ds(NS * ROWS_PER_TILE, TAIL_ROWS)],
            out_hbm.at[pl.ds(c * N + NS * ROWS_PER_TILE, TAIL_ROWS)])


def _sc_degree(edst_p, ones8, deg_init):
    mesh = plsc.VectorSubcoreMesh(core_axis_name="c", subcore_axis_name="s")
    per_tile = NCHUNK // (NC * NS)
    run = pl.kernel(
        _sc_degree_body,
        out_type=jax.ShapeDtypeStruct((2 * N, 8), jnp.float32),
        mesh=mesh,
        scratch_types=[
            pltpu.VMEM((per_tile, CHUNK), jnp.int32),
            pltpu.VMEM((CHUNK, 8), jnp.float32),
            pltpu.VMEM_SHARED((ACC_ROWS, 8), jnp.float32),
            pltpu.SemaphoreType.DMA,
        ],
    )
    return run(edst_p, ones8, deg_init)


def _sc_scatter_body(g_hbm, esrc_hbm, edst_hbm, out_hbm, src_v, dst_v, rows_v,
                     acc_sh, sem):
    c = lax.axis_index("c")
    s = lax.axis_index("s")
    rbase = s * ROWS_PER_TILE
    pltpu.sync_copy(g_hbm.at[pl.ds(c * N + rbase, ROWS_PER_TILE)],
                    acc_sh.at[pl.ds(rbase, ROWS_PER_TILE)])
    @pl.when(s == NS - 1)
    def _():
        pltpu.sync_copy(
            g_hbm.at[pl.ds(c * N + NS * ROWS_PER_TILE, TAIL_ROWS)],
            acc_sh.at[pl.ds(NS * ROWS_PER_TILE, TAIL_ROWS)])

    rows0, rows1 = rows_v
    sem0, sem1 = sem
    per_tile = NCHUNK // NS
    cbase = s * per_tile
    off = c * N
    plsc.subcore_barrier()

    def run_stage(st, carry):
        sbase = cbase + st * IDX_STAGE
        pltpu.sync_copy(esrc_hbm.at[pl.ds(sbase, IDX_STAGE)], src_v)
        pltpu.sync_copy(edst_hbm.at[pl.ds(sbase, IDX_STAGE)], dst_v)

        def add_off(k, carry2):
            r = k // (CHUNK // LANES)
            j = k % (CHUNK // LANES)
            src_v[r, pl.ds(j * LANES, LANES)] = (
                src_v[r, pl.ds(j * LANES, LANES)] + off)
            return carry2
        lax.fori_loop(0, IDX_STAGE * (CHUNK // LANES), add_off, 0)

        pltpu.async_copy(g_hbm.at[src_v.at[0]], rows0, sem0)

        def chunk(k, carry2):
            @pl.when(k + 1 < IDX_STAGE)
            def _():
                @pl.when((k + 1) % 2 == 0)
                def _():
                    pltpu.async_copy(g_hbm.at[src_v.at[k + 1]], rows0, sem0)
                @pl.when((k + 1) % 2 == 1)
                def _():
                    pltpu.async_copy(g_hbm.at[src_v.at[k + 1]], rows1, sem1)

            @pl.when(k % 2 == 0)
            def _():
                pltpu.make_async_copy(g_hbm.at[src_v.at[k]], rows0, sem0).wait()
                pltpu.sync_copy(rows0, acc_sh.at[dst_v.at[k]], add=True)
            @pl.when(k % 2 == 1)
            def _():
                pltpu.make_async_copy(g_hbm.at[src_v.at[k]], rows1, sem1).wait()
                pltpu.sync_copy(rows1, acc_sh.at[dst_v.at[k]], add=True)
            return carry2
        lax.fori_loop(0, IDX_STAGE, chunk, 0)
        return carry
    lax.fori_loop(0, per_tile // IDX_STAGE, run_stage, 0)

    plsc.subcore_barrier()
    pltpu.sync_copy(acc_sh.at[pl.ds(rbase, ROWS_PER_TILE)],
                    out_hbm.at[pl.ds(c * N + rbase, ROWS_PER_TILE)])
    @pl.when(s == NS - 1)
    def _():
        pltpu.sync_copy(
            acc_sh.at[pl.ds(NS * ROWS_PER_TILE, TAIL_ROWS)],
            out_hbm.at[pl.ds(c * N + NS * ROWS_PER_TILE, TAIL_ROWS)])


def _sc_scatter(g_stack, esrc_p, edst_p):
    mesh = plsc.VectorSubcoreMesh(core_axis_name="c", subcore_axis_name="s")
    run = pl.kernel(
        _sc_scatter_body,
        out_type=jax.ShapeDtypeStruct((2 * N, HALF), jnp.float32),
        mesh=mesh,
        scratch_types=[
            pltpu.VMEM((IDX_STAGE, CHUNK), jnp.int32),
            pltpu.VMEM((IDX_STAGE, CHUNK), jnp.int32),
            (pltpu.VMEM((CHUNK, HALF), jnp.float32),
             pltpu.VMEM((CHUNK, HALF), jnp.float32)),
            pltpu.VMEM_SHARED((ACC_ROWS, HALF), jnp.float32),
            (pltpu.SemaphoreType.DMA, pltpu.SemaphoreType.DMA),
        ],
    )
    return run(g_stack, esrc_p, edst_p)


def _mm1_body(x_ref, w_ref, dga_ref, dgb_ref, g_ref, dinv_ref):
    deg = dga_ref[:, 0:1] + dgb_ref[:, 0:1]
    dinv = lax.rsqrt(deg)
    h = jnp.dot(x_ref[...], w_ref[...], preferred_element_type=jnp.float32)
    g_ref[...] = h * dinv
    dinv_ref[...] = dinv


def _mm1(x, W1, deg_parts):
    return pl.pallas_call(
        _mm1_body,
        grid=(NB, 2),
        in_specs=[
            pl.BlockSpec((RB, D_IN), lambda i, j: (i, 0)),
            pl.BlockSpec((D_IN, HALF), lambda i, j: (0, j)),
            pl.BlockSpec((RB, 8), lambda i, j: (i, 0)),
            pl.BlockSpec((RB, 8), lambda i, j: (NB + i, 0)),
        ],
        out_specs=[
            pl.BlockSpec((RB, HALF), lambda i, j: (j * NB + i, 0)),
            pl.BlockSpec((RB, 1), lambda i, j: (i, 0)),
        ],
        out_shape=[
            jax.ShapeDtypeStruct((2 * N, HALF), jnp.float32),
            jax.ShapeDtypeStruct((N, 1), jnp.float32),
        ],
        compiler_params=pltpu.CompilerParams(
            dimension_semantics=("arbitrary", "arbitrary")),
    )(x, W1, deg_parts, deg_parts)


def _mm2_body(alo_ref, ahi_ref, dinv_ref, b1_ref, w2_ref, g2_ref):
    h = jnp.concatenate([alo_ref[...], ahi_ref[...]], axis=1)
    h = jnp.maximum(h * dinv_ref[...] + b1_ref[...], 0.0)
    g2 = jnp.dot(h, w2_ref[...], preferred_element_type=jnp.float32)
    g2_ref[...] = g2 * dinv_ref[...]


def _mm2(acc1, dinv, b1r, W2):
    return pl.pallas_call(
        _mm2_body,
        grid=(NB, 2),
        in_specs=[
            pl.BlockSpec((RB, HALF), lambda i, j: (i, 0)),
            pl.BlockSpec((RB, HALF), lambda i, j: (NB + i, 0)),
            pl.BlockSpec((RB, 1), lambda i, j: (i, 0)),
            pl.BlockSpec((1, D_H), lambda i, j: (0, 0)),
            pl.BlockSpec((D_H, HALF), lambda i, j: (0, j)),
        ],
        out_specs=pl.BlockSpec((RB, HALF), lambda i, j: (j * NB + i, 0)),
        out_shape=jax.ShapeDtypeStruct((2 * N, HALF), jnp.float32),
        compiler_params=pltpu.CompilerParams(
            dimension_semantics=("arbitrary", "arbitrary")),
    )(acc1, acc1, dinv, b1r, W2)


def _final_body(alo_ref, ahi_ref, dinv_ref, b2_ref, wfc_ref, bfc_ref,
                batch_ref, out_ref, sacc, cacc):
    i = pl.program_id(0)
    h = jnp.concatenate([alo_ref[...], ahi_ref[...]], axis=1)
    h = jnp.maximum(h * dinv_ref[...] + b2_ref[...], 0.0)
    y = jnp.dot(h, wfc_ref[...], preferred_element_type=jnp.float32)
    y = y + bfc_ref[0, 0]
    bb = batch_ref[0, 0, :]
    onehot = (bb[:, None] == lax.broadcasted_iota(jnp.int32, (1, G), 1)
              ).astype(jnp.float32)
    ps = jnp.sum(onehot * y, axis=0)[:, None]
    cs = jnp.sum(onehot, axis=0)[:, None]

    @pl.when(i == 0)
    def _():
        sacc[...] = jnp.zeros_like(sacc)
        cacc[...] = jnp.zeros_like(cacc)

    sacc[...] += ps
    cacc[...] += cs

    @pl.when(i == NB - 1)
    def _():
        out_ref[...] = sacc[...] / jnp.maximum(cacc[...], 1.0)


def _final(acc2, dinv, b2r, Wfc, bfcr, batch3):
    return pl.pallas_call(
        _final_body,
        grid=(NB,),
        in_specs=[
            pl.BlockSpec((RB, HALF), lambda i: (i, 0)),
            pl.BlockSpec((RB, HALF), lambda i: (NB + i, 0)),
            pl.BlockSpec((RB, 1), lambda i: (i, 0)),
            pl.BlockSpec((1, D_H), lambda i: (0, 0)),
            pl.BlockSpec((D_H, 1), lambda i: (0, 0)),
            pl.BlockSpec((1, 1), lambda i: (0, 0)),
            pl.BlockSpec((1, 1, RB), lambda i: (i, 0, 0)),
        ],
        out_specs=pl.BlockSpec((G, 1), lambda i: (0, 0)),
        out_shape=jax.ShapeDtypeStruct((G, 1), jnp.float32),
        scratch_shapes=[
            pltpu.VMEM((G, 1), jnp.float32),
            pltpu.VMEM((G, 1), jnp.float32),
        ],
        compiler_params=pltpu.CompilerParams(
            dimension_semantics=("arbitrary",)),
    )(acc2, acc2, dinv, b2r, Wfc, bfcr, batch3)


def kernel(x, edge_index, batch, W1, b1, W2, b2, Wfc, bfc):
    esrc = edge_index[0]
    edst = edge_index[1]
    pad = E_PAD - E
    esrc_p = jnp.concatenate(
        [esrc, jnp.zeros((pad,), jnp.int32)]).reshape(NCHUNK, CHUNK)
    edst_p = jnp.concatenate(
        [edst, jnp.full((pad,), N, jnp.int32)]).reshape(NCHUNK, CHUNK)

    ones8 = jnp.ones((CHUNK, 8), jnp.float32)
    deg_init = jnp.concatenate([
        jnp.ones((ACC_ROWS, 8), jnp.float32),
        jnp.zeros((ACC_ROWS, 8), jnp.float32)])

    b1r = b1.reshape(1, D_H)
    b2r = b2.reshape(1, D_H)
    bfcr = bfc.reshape(1, 1)
    batch3 = batch.reshape(NB, 1, RB)

    deg_parts = _sc_degree(edst_p, ones8, deg_init)
    g1, dinv = _mm1(x, W1, deg_parts)
    acc1 = _sc_scatter(g1, esrc_p, edst_p)
    g2 = _mm2(acc1, dinv, b1r, W2)
    acc2 = _sc_scatter(g2, esrc_p, edst_p)
    return _final(acc2, dinv, b2r, Wfc, bfcr, batch3)


# speedup vs baseline: 23.2220x; 2.2075x over previous
import functools

import jax
import jax.numpy as jnp
from jax import lax
from jax.experimental import pallas as pl
from jax.experimental.pallas import tpu as pltpu
from jax.experimental.pallas import tpu_sc as plsc

N = 10000
E = 320000
G = 64
D_IN = 128
D_H = 256
HALF = 128

NC = 2
NS = 16
LANES = 16

CHUNK = 128
IDX_STAGE = 40
E_PAD = 327680
NCHUNK = E_PAD // CHUNK
ROWS_PER_TILE = 624
TAIL_ROWS = N - NS * ROWS_PER_TILE
ACC_ROWS = N + LANES

RB = 2000
NB = N // RB


def _sc_degree_body(edst_hbm, ones_hbm, init_hbm, out_hbm, dst_v, ones_v,
                    deg_sh, sem):
    c = lax.axis_index("c")
    s = lax.axis_index("s")
    rbase = s * ROWS_PER_TILE
    pltpu.sync_copy(init_hbm.at[pl.ds(c * ACC_ROWS + rbase, ROWS_PER_TILE)],
                    deg_sh.at[pl.ds(rbase, ROWS_PER_TILE)])
    @pl.when(s == NS - 1)
    def _():
        pltpu.sync_copy(
            init_hbm.at[pl.ds(c * ACC_ROWS + NS * ROWS_PER_TILE,
                              TAIL_ROWS + LANES)],
            deg_sh.at[pl.ds(NS * ROWS_PER_TILE, TAIL_ROWS + LANES)])
    pltpu.sync_copy(ones_hbm, ones_v)

    per_tile = NCHUNK // (NC * NS)
    cbase = (c * NS + s) * per_tile
    pltpu.sync_copy(edst_hbm.at[pl.ds(cbase, per_tile)], dst_v)
    plsc.subcore_barrier()

    def chunk(k, carry):
        pltpu.sync_copy(ones_v, deg_sh.at[dst_v.at[k]], add=True)
        return carry
    lax.fori_loop(0, per_tile, chunk, 0)

    plsc.subcore_barrier()
    pltpu.sync_copy(deg_sh.at[pl.ds(rbase, ROWS_PER_TILE)],
                    out_hbm.at[pl.ds(c * N + rbase, ROWS_PER_TILE)])
    @pl.when(s == NS - 1)
    def _():
        pltpu.sync_copy(
            deg_sh.at[pl.ds(NS * ROWS_PER_TILE, TAIL_ROWS)],
            out_hbm.at[pl.ds(c * N + NS * ROWS_PER_TILE, TAIL_ROWS)])


def _sc_degree(edst_p, ones8, deg_init):
    mesh = plsc.VectorSubcoreMesh(core_axis_name="c", subcore_axis_name="s")
    per_tile = NCHUNK // (NC * NS)
    run = pl.kernel(
        _sc_degree_body,
        out_type=jax.ShapeDtypeStruct((2 * N, 8), jnp.float32),
        mesh=mesh,
        scratch_types=[
            pltpu.VMEM((per_tile, CHUNK), jnp.int32),
            pltpu.VMEM((CHUNK, 8), jnp.float32),
            pltpu.VMEM_SHARED((ACC_ROWS, 8), jnp.float32),
            pltpu.SemaphoreType.DMA,
        ],
    )
    return run(edst_p, ones8, deg_init)


def _sc_scatter_body(g_hbm, esrc_hbm, edst_hbm, out_hbm, src_v, dst_v, rows_v,
                     acc_sh, sem):
    c = lax.axis_index("c")
    s = lax.axis_index("s")
    rbase = s * ROWS_PER_TILE
    pltpu.sync_copy(g_hbm.at[pl.ds(c * N + rbase, ROWS_PER_TILE)],
                    acc_sh.at[pl.ds(rbase, ROWS_PER_TILE)])
    @pl.when(s == NS - 1)
    def _():
        pltpu.sync_copy(
            g_hbm.at[pl.ds(c * N + NS * ROWS_PER_TILE, TAIL_ROWS)],
            acc_sh.at[pl.ds(NS * ROWS_PER_TILE, TAIL_ROWS)])

    rows0, rows1 = rows_v
    gsem0, gsem1 = sem
    per_tile = NCHUNK // NS
    cbase = s * per_tile
    plsc.subcore_barrier()

    def run_stage(st, carry):
        sbase = cbase + st * IDX_STAGE
        pltpu.sync_copy(esrc_hbm.at[pl.ds(c * NCHUNK + sbase, IDX_STAGE)],
                        src_v)
        pltpu.sync_copy(edst_hbm.at[pl.ds(sbase, IDX_STAGE)], dst_v)

        pltpu.async_copy(g_hbm.at[src_v.at[0]], rows0, gsem0)

        def chunk(kk, carry2):
            k0 = 2 * kk
            pltpu.async_copy(g_hbm.at[src_v.at[k0 + 1]], rows1, gsem1)
            pltpu.make_async_copy(g_hbm.at[src_v.at[k0]], rows0, gsem0).wait()
            pltpu.sync_copy(rows0, acc_sh.at[dst_v.at[k0]], add=True)
            @pl.when(k0 + 2 < IDX_STAGE)
            def _():
                pltpu.async_copy(g_hbm.at[src_v.at[k0 + 2]], rows0, gsem0)
            pltpu.make_async_copy(g_hbm.at[src_v.at[k0 + 1]], rows1,
                                  gsem1).wait()
            pltpu.sync_copy(rows1, acc_sh.at[dst_v.at[k0 + 1]], add=True)
            return carry2
        lax.fori_loop(0, IDX_STAGE // 2, chunk, 0)
        return carry
    lax.fori_loop(0, per_tile // IDX_STAGE, run_stage, 0)

    plsc.subcore_barrier()
    pltpu.sync_copy(acc_sh.at[pl.ds(rbase, ROWS_PER_TILE)],
                    out_hbm.at[pl.ds(c * N + rbase, ROWS_PER_TILE)])
    @pl.when(s == NS - 1)
    def _():
        pltpu.sync_copy(
            acc_sh.at[pl.ds(NS * ROWS_PER_TILE, TAIL_ROWS)],
            out_hbm.at[pl.ds(c * N + NS * ROWS_PER_TILE, TAIL_ROWS)])


def _sc_scatter(g_stack, esrc2, edst_p):
    mesh = plsc.VectorSubcoreMesh(core_axis_name="c", subcore_axis_name="s")
    run = pl.kernel(
        _sc_scatter_body,
        out_type=jax.ShapeDtypeStruct((2 * N, HALF), jnp.float32),
        mesh=mesh,
        scratch_types=[
            pltpu.VMEM((IDX_STAGE, CHUNK), jnp.int32),
            pltpu.VMEM((IDX_STAGE, CHUNK), jnp.int32),
            (pltpu.VMEM((CHUNK, HALF), jnp.float32),
             pltpu.VMEM((CHUNK, HALF), jnp.float32)),
            pltpu.VMEM_SHARED((ACC_ROWS, HALF), jnp.float32),
            (pltpu.SemaphoreType.DMA, pltpu.SemaphoreType.DMA),
        ],
    )
    return run(g_stack, esrc2, edst_p)


def _mm1_body(x_ref, w_ref, dga_ref, dgb_ref, g_ref, dinv_ref):
    deg = dga_ref[:, 0:1] + dgb_ref[:, 0:1]
    dinv = lax.rsqrt(deg)
    h = jnp.dot(x_ref[...], w_ref[...], preferred_element_type=jnp.float32)
    g_ref[...] = h * dinv
    dinv_ref[...] = dinv


def _mm1(x, W1, deg_parts):
    return pl.pallas_call(
        _mm1_body,
        grid=(NB, 2),
        in_specs=[
            pl.BlockSpec((RB, D_IN), lambda i, j: (i, 0)),
            pl.BlockSpec((D_IN, HALF), lambda i, j: (0, j)),
            pl.BlockSpec((RB, 8), lambda i, j: (i, 0)),
            pl.BlockSpec((RB, 8), lambda i, j: (NB + i, 0)),
        ],
        out_specs=[
            pl.BlockSpec((RB, HALF), lambda i, j: (j * NB + i, 0)),
            pl.BlockSpec((RB, 1), lambda i, j: (i, 0)),
        ],
        out_shape=[
            jax.ShapeDtypeStruct((2 * N, HALF), jnp.float32),
            jax.ShapeDtypeStruct((N, 1), jnp.float32),
        ],
        compiler_params=pltpu.CompilerParams(
            dimension_semantics=("arbitrary", "arbitrary")),
    )(x, W1, deg_parts, deg_parts)


def _mm2_body(alo_ref, ahi_ref, dinv_ref, b1_ref, w2_ref, g2_ref):
    h = jnp.concatenate([alo_ref[...], ahi_ref[...]], axis=1)
    h = jnp.maximum(h * dinv_ref[...] + b1_ref[...], 0.0)
    g2 = jnp.dot(h, w2_ref[...], preferred_element_type=jnp.float32)
    g2_ref[...] = g2 * dinv_ref[...]


def _mm2(acc1, dinv, b1r, W2):
    return pl.pallas_call(
        _mm2_body,
        grid=(NB, 2),
        in_specs=[
            pl.BlockSpec((RB, HALF), lambda i, j: (i, 0)),
            pl.BlockSpec((RB, HALF), lambda i, j: (NB + i, 0)),
            pl.BlockSpec((RB, 1), lambda i, j: (i, 0)),
            pl.BlockSpec((1, D_H), lambda i, j: (0, 0)),
            pl.BlockSpec((D_H, HALF), lambda i, j: (0, j)),
        ],
        out_specs=pl.BlockSpec((RB, HALF), lambda i, j: (j * NB + i, 0)),
        out_shape=jax.ShapeDtypeStruct((2 * N, HALF), jnp.float32),
        compiler_params=pltpu.CompilerParams(
            dimension_semantics=("arbitrary", "arbitrary")),
    )(acc1, acc1, dinv, b1r, W2)


def _final_body(alo_ref, ahi_ref, dinv_ref, b2_ref, wfc_ref, bfc_ref,
                batch_ref, out_ref, sacc, cacc):
    i = pl.program_id(0)
    h = jnp.concatenate([alo_ref[...], ahi_ref[...]], axis=1)
    h = jnp.maximum(h * dinv_ref[...] + b2_ref[...], 0.0)
    y = jnp.dot(h, wfc_ref[...], preferred_element_type=jnp.float32)
    y = y + bfc_ref[0, 0]
    bb = batch_ref[0, 0, :]
    onehot = (bb[:, None] == lax.broadcasted_iota(jnp.int32, (1, G), 1)
              ).astype(jnp.float32)
    ps = jnp.sum(onehot * y, axis=0)[:, None]
    cs = jnp.sum(onehot, axis=0)[:, None]

    @pl.when(i == 0)
    def _():
        sacc[...] = jnp.zeros_like(sacc)
        cacc[...] = jnp.zeros_like(cacc)

    sacc[...] += ps
    cacc[...] += cs

    @pl.when(i == NB - 1)
    def _():
        out_ref[...] = sacc[...] / jnp.maximum(cacc[...], 1.0)


def _final(acc2, dinv, b2r, Wfc, bfcr, batch3):
    return pl.pallas_call(
        _final_body,
        grid=(NB,),
        in_specs=[
            pl.BlockSpec((RB, HALF), lambda i: (i, 0)),
            pl.BlockSpec((RB, HALF), lambda i: (NB + i, 0)),
            pl.BlockSpec((RB, 1), lambda i: (i, 0)),
            pl.BlockSpec((1, D_H), lambda i: (0, 0)),
            pl.BlockSpec((D_H, 1), lambda i: (0, 0)),
            pl.BlockSpec((1, 1), lambda i: (0, 0)),
            pl.BlockSpec((1, 1, RB), lambda i: (i, 0, 0)),
        ],
        out_specs=pl.BlockSpec((G, 1), lambda i: (0, 0)),
        out_shape=jax.ShapeDtypeStruct((G, 1), jnp.float32),
        scratch_shapes=[
            pltpu.VMEM((G, 1), jnp.float32),
            pltpu.VMEM((G, 1), jnp.float32),
        ],
        compiler_params=pltpu.CompilerParams(
            dimension_semantics=("arbitrary",)),
    )(acc2, acc2, dinv, b2r, Wfc, bfcr, batch3)


def kernel(x, edge_index, batch, W1, b1, W2, b2, Wfc, bfc):
    esrc = edge_index[0]
    edst = edge_index[1]
    pad = E_PAD - E
    pad_src = (jnp.arange(pad, dtype=jnp.int32) * 131) % N
    pad_dst = N + (jnp.arange(pad, dtype=jnp.int32) % LANES)
    esrc_p = jnp.concatenate([esrc, pad_src]).reshape(NCHUNK, CHUNK)
    edst_p = jnp.concatenate([edst, pad_dst]).reshape(NCHUNK, CHUNK)
    esrc2 = jnp.concatenate([esrc_p, esrc_p + N])

    ones8 = jnp.ones((CHUNK, 8), jnp.float32)
    deg_init = jnp.concatenate([
        jnp.ones((ACC_ROWS, 8), jnp.float32),
        jnp.zeros((ACC_ROWS, 8), jnp.float32)])

    b1r = b1.reshape(1, D_H)
    b2r = b2.reshape(1, D_H)
    bfcr = bfc.reshape(1, 1)
    batch3 = batch.reshape(NB, 1, RB)

    deg_parts = _sc_degree(edst_p, ones8, deg_init)
    g1, dinv = _mm1(x, W1, deg_parts)
    acc1 = _sc_scatter(g1, esrc2, edst_p)
    g2 = _mm2(acc1, dinv, b1r, W2)
    acc2 = _sc_scatter(g2, esrc2, edst_p)
    return _final(acc2, dinv, b2r, Wfc, bfcr, batch3)


# speedup vs baseline: 28.1971x; 1.2142x over previous
import functools

import jax
import jax.numpy as jnp
from jax import lax
from jax.experimental import pallas as pl
from jax.experimental.pallas import tpu as pltpu
from jax.experimental.pallas import tpu_sc as plsc

N = 10000
E = 320000
G = 64
D_IN = 128
D_H = 256
HALF = 128

NC = 2
NS = 16
LANES = 16

CHUNK = 128
IDX_STAGE = 40
E_PAD = 327680
NCHUNK = E_PAD // CHUNK
ROWS_PER_TILE = 624
TAIL_ROWS = N - NS * ROWS_PER_TILE
ACC_ROWS = N + LANES

RB = 2000
NB = N // RB


def _sc_degree_body(edst_hbm, ones_hbm, init_hbm, out_hbm, dst_v, ones_v,
                    deg_sh, sem):
    c = lax.axis_index("c")
    s = lax.axis_index("s")
    rbase = s * ROWS_PER_TILE
    pltpu.sync_copy(init_hbm.at[pl.ds(c * ACC_ROWS + rbase, ROWS_PER_TILE)],
                    deg_sh.at[pl.ds(rbase, ROWS_PER_TILE)])
    @pl.when(s == NS - 1)
    def _():
        pltpu.sync_copy(
            init_hbm.at[pl.ds(c * ACC_ROWS + NS * ROWS_PER_TILE,
                              TAIL_ROWS + LANES)],
            deg_sh.at[pl.ds(NS * ROWS_PER_TILE, TAIL_ROWS + LANES)])
    pltpu.sync_copy(ones_hbm, ones_v)

    per_tile = NCHUNK // (NC * NS)
    cbase = (c * NS + s) * per_tile
    pltpu.sync_copy(edst_hbm.at[pl.ds(cbase, per_tile)], dst_v)
    plsc.subcore_barrier()

    def chunk(k, carry):
        pltpu.sync_copy(ones_v, deg_sh.at[dst_v.at[k]], add=True)
        return carry
    lax.fori_loop(0, per_tile, chunk, 0)

    plsc.subcore_barrier()
    pltpu.sync_copy(deg_sh.at[pl.ds(rbase, ROWS_PER_TILE)],
                    out_hbm.at[pl.ds(c * N + rbase, ROWS_PER_TILE)])
    @pl.when(s == NS - 1)
    def _():
        pltpu.sync_copy(
            deg_sh.at[pl.ds(NS * ROWS_PER_TILE, TAIL_ROWS)],
            out_hbm.at[pl.ds(c * N + NS * ROWS_PER_TILE, TAIL_ROWS)])


def _sc_degree(edst_p, ones8, deg_init):
    mesh = plsc.VectorSubcoreMesh(core_axis_name="c", subcore_axis_name="s")
    per_tile = NCHUNK // (NC * NS)
    run = pl.kernel(
        _sc_degree_body,
        out_type=jax.ShapeDtypeStruct((2 * N, 8), jnp.float32),
        mesh=mesh,
        scratch_types=[
            pltpu.VMEM((per_tile, CHUNK), jnp.int32),
            pltpu.VMEM((CHUNK, 8), jnp.float32),
            pltpu.VMEM_SHARED((ACC_ROWS, 8), jnp.float32),
            pltpu.SemaphoreType.DMA,
        ],
    )
    return run(edst_p, ones8, deg_init)


def _sc_scatter_body(g_hbm, esrc_hbm, edst_hbm, out_hbm, src_v, dst_v, rows_v,
                     acc_sh, sem, *, edge_split):
    c = lax.axis_index("c")
    s = lax.axis_index("s")
    rbase = s * ROWS_PER_TILE
    pltpu.sync_copy(g_hbm.at[pl.ds(c * N + rbase, ROWS_PER_TILE)],
                    acc_sh.at[pl.ds(rbase, ROWS_PER_TILE)])
    @pl.when(s == NS - 1)
    def _():
        pltpu.sync_copy(
            g_hbm.at[pl.ds(c * N + NS * ROWS_PER_TILE, TAIL_ROWS)],
            acc_sh.at[pl.ds(NS * ROWS_PER_TILE, TAIL_ROWS)])

    rows0, rows1 = rows_v
    gsem0, gsem1 = sem
    if edge_split:
        per_tile = NCHUNK // (NC * NS)
        cbase = (c * NS + s) * per_tile
    else:
        per_tile = NCHUNK // NS
        cbase = s * per_tile
    plsc.subcore_barrier()

    def run_stage(st, carry):
        sbase = cbase + st * IDX_STAGE
        pltpu.sync_copy(esrc_hbm.at[pl.ds(c * NCHUNK + sbase, IDX_STAGE)],
                        src_v)
        pltpu.sync_copy(edst_hbm.at[pl.ds(sbase, IDX_STAGE)], dst_v)

        pltpu.async_copy(g_hbm.at[src_v.at[0]], rows0, gsem0)

        def chunk(kk, carry2):
            k0 = 2 * kk
            pltpu.async_copy(g_hbm.at[src_v.at[k0 + 1]], rows1, gsem1)
            pltpu.make_async_copy(g_hbm.at[src_v.at[k0]], rows0, gsem0).wait()
            pltpu.sync_copy(rows0, acc_sh.at[dst_v.at[k0]], add=True)
            @pl.when(k0 + 2 < IDX_STAGE)
            def _():
                pltpu.async_copy(g_hbm.at[src_v.at[k0 + 2]], rows0, gsem0)
            pltpu.make_async_copy(g_hbm.at[src_v.at[k0 + 1]], rows1,
                                  gsem1).wait()
            pltpu.sync_copy(rows1, acc_sh.at[dst_v.at[k0 + 1]], add=True)
            return carry2
        lax.fori_loop(0, IDX_STAGE // 2, chunk, 0)
        return carry
    lax.fori_loop(0, per_tile // IDX_STAGE, run_stage, 0)

    plsc.subcore_barrier()
    pltpu.sync_copy(acc_sh.at[pl.ds(rbase, ROWS_PER_TILE)],
                    out_hbm.at[pl.ds(c * N + rbase, ROWS_PER_TILE)])
    @pl.when(s == NS - 1)
    def _():
        pltpu.sync_copy(
            acc_sh.at[pl.ds(NS * ROWS_PER_TILE, TAIL_ROWS)],
            out_hbm.at[pl.ds(c * N + NS * ROWS_PER_TILE, TAIL_ROWS)])


def _sc_scatter(g_stack, esrc2, edst_p, width, edge_split=False):
    mesh = plsc.VectorSubcoreMesh(core_axis_name="c", subcore_axis_name="s")
    run = pl.kernel(
        functools.partial(_sc_scatter_body, edge_split=edge_split),
        out_type=jax.ShapeDtypeStruct((2 * N, width), jnp.float32),
        mesh=mesh,
        scratch_types=[
            pltpu.VMEM((IDX_STAGE, CHUNK), jnp.int32),
            pltpu.VMEM((IDX_STAGE, CHUNK), jnp.int32),
            (pltpu.VMEM((CHUNK, width), jnp.float32),
             pltpu.VMEM((CHUNK, width), jnp.float32)),
            pltpu.VMEM_SHARED((ACC_ROWS, width), jnp.float32),
            (pltpu.SemaphoreType.DMA, pltpu.SemaphoreType.DMA),
        ],
    )
    return run(g_stack, esrc2, edst_p)


def _scale1_body(x_ref, dga_ref, dgb_ref, t_ref, dinv_ref):
    deg = dga_ref[:, 0:1] + dgb_ref[:, 0:1]
    dinv = lax.rsqrt(deg)
    t_ref[...] = x_ref[...] * dinv
    dinv_ref[...] = dinv


def _scale1(x, deg_parts):
    return pl.pallas_call(
        _scale1_body,
        grid=(NB, 2),
        in_specs=[
            pl.BlockSpec((RB, D_IN), lambda i, j: (i, 0)),
            pl.BlockSpec((RB, 8), lambda i, j: (i, 0)),
            pl.BlockSpec((RB, 8), lambda i, j: (NB + i, 0)),
        ],
        out_specs=[
            pl.BlockSpec((RB, D_IN), lambda i, j: (j * NB + i, 0)),
            pl.BlockSpec((RB, 1), lambda i, j: (i, 0)),
        ],
        out_shape=[
            jax.ShapeDtypeStruct((2 * N, D_IN), jnp.float32),
            jax.ShapeDtypeStruct((N, 1), jnp.float32),
        ],
        compiler_params=pltpu.CompilerParams(
            dimension_semantics=("arbitrary", "arbitrary")),
    )(x, deg_parts, deg_parts)


def _tca_body(apa_ref, apb_ref, x_ref, dinv_ref, w1_ref, b1_ref, t2_ref):
    dinv = dinv_ref[...]
    acc = apa_ref[...] + apb_ref[...] - x_ref[...] * dinv
    h = jnp.dot(acc, w1_ref[...], preferred_element_type=jnp.float32)
    h = jnp.maximum(h * dinv + b1_ref[...], 0.0)
    t2_ref[...] = h * dinv


def _tca(acc1, x, dinv, W1, b1r):
    return pl.pallas_call(
        _tca_body,
        grid=(NB, 2),
        in_specs=[
            pl.BlockSpec((RB, D_IN), lambda i, j: (i, 0)),
            pl.BlockSpec((RB, D_IN), lambda i, j: (NB + i, 0)),
            pl.BlockSpec((RB, D_IN), lambda i, j: (i, 0)),
            pl.BlockSpec((RB, 1), lambda i, j: (i, 0)),
            pl.BlockSpec((D_IN, HALF), lambda i, j: (0, j)),
            pl.BlockSpec((1, HALF), lambda i, j: (0, j)),
        ],
        out_specs=pl.BlockSpec((RB, HALF), lambda i, j: (j * NB + i, 0)),
        out_shape=jax.ShapeDtypeStruct((2 * N, HALF), jnp.float32),
        compiler_params=pltpu.CompilerParams(
            dimension_semantics=("arbitrary", "arbitrary")),
    )(acc1, acc1, x, dinv, W1, b1r)


def _tcb_body(alo_ref, ahi_ref, dinv_ref, w2_ref, b2_ref, wfc_ref, bfc_ref,
              batch_ref, out_ref, sacc, cacc):
    i = pl.program_id(0)
    acc = jnp.concatenate([alo_ref[...], ahi_ref[...]], axis=1)
    h = jnp.dot(acc, w2_ref[...], preferred_element_type=jnp.float32)
    h = jnp.maximum(h * dinv_ref[...] + b2_ref[...], 0.0)
    y = jnp.dot(h, wfc_ref[...], preferred_element_type=jnp.float32)
    y = y + bfc_ref[0, 0]
    bb = batch_ref[0, 0, :]
    onehot = (bb[:, None] == lax.broadcasted_iota(jnp.int32, (1, G), 1)
              ).astype(jnp.float32)
    ps = jnp.sum(onehot * y, axis=0)[:, None]
    cs = jnp.sum(onehot, axis=0)[:, None]

    @pl.when(i == 0)
    def _():
        sacc[...] = jnp.zeros_like(sacc)
        cacc[...] = jnp.zeros_like(cacc)

    sacc[...] += ps
    cacc[...] += cs

    @pl.when(i == NB - 1)
    def _():
        out_ref[...] = sacc[...] / jnp.maximum(cacc[...], 1.0)


def _tcb(acc2, dinv, W2, b2r, Wfc, bfcr, batch3):
    return pl.pallas_call(
        _tcb_body,
        grid=(NB,),
        in_specs=[
            pl.BlockSpec((RB, HALF), lambda i: (i, 0)),
            pl.BlockSpec((RB, HALF), lambda i: (NB + i, 0)),
            pl.BlockSpec((RB, 1), lambda i: (i, 0)),
            pl.BlockSpec((D_H, D_H), lambda i: (0, 0)),
            pl.BlockSpec((1, D_H), lambda i: (0, 0)),
            pl.BlockSpec((D_H, 1), lambda i: (0, 0)),
            pl.BlockSpec((1, 1), lambda i: (0, 0)),
            pl.BlockSpec((1, 1, RB), lambda i: (i, 0, 0)),
        ],
        out_specs=pl.BlockSpec((G, 1), lambda i: (0, 0)),
        out_shape=jax.ShapeDtypeStruct((G, 1), jnp.float32),
        scratch_shapes=[
            pltpu.VMEM((G, 1), jnp.float32),
            pltpu.VMEM((G, 1), jnp.float32),
        ],
        compiler_params=pltpu.CompilerParams(
            dimension_semantics=("arbitrary",)),
    )(acc2, acc2, dinv, W2, b2r, Wfc, bfcr, batch3)


def kernel(x, edge_index, batch, W1, b1, W2, b2, Wfc, bfc):
    esrc = edge_index[0]
    edst = edge_index[1]
    pad = E_PAD - E
    pad_src = (jnp.arange(pad, dtype=jnp.int32) * 131) % N
    pad_dst = N + (jnp.arange(pad, dtype=jnp.int32) % LANES)
    esrc_p = jnp.concatenate([esrc, pad_src]).reshape(NCHUNK, CHUNK)
    edst_p = jnp.concatenate([edst, pad_dst]).reshape(NCHUNK, CHUNK)
    esrc2 = jnp.concatenate([esrc_p, esrc_p + N])

    ones8 = jnp.ones((CHUNK, 8), jnp.float32)
    deg_init = jnp.concatenate([
        jnp.ones((ACC_ROWS, 8), jnp.float32),
        jnp.zeros((ACC_ROWS, 8), jnp.float32)])

    b1r = b1.reshape(1, D_H)
    b2r = b2.reshape(1, D_H)
    bfcr = bfc.reshape(1, 1)
    batch3 = batch.reshape(NB, 1, RB)

    deg_parts = _sc_degree(edst_p, ones8, deg_init)
    t1d, dinv = _scale1(x, deg_parts)
    acc1 = _sc_scatter(t1d, esrc2, edst_p, D_IN, edge_split=True)
    t2 = _tca(acc1, x, dinv, W1, b1r)
    acc2 = _sc_scatter(t2, esrc2, edst_p, HALF)
    return _tcb(acc2, dinv, W2, b2r, Wfc, bfcr, batch3)


# speedup vs baseline: 28.5014x; 1.0108x over previous
import functools

import jax
import jax.numpy as jnp
from jax import lax
from jax.experimental import pallas as pl
from jax.experimental.pallas import tpu as pltpu
from jax.experimental.pallas import tpu_sc as plsc

N = 10000
E = 320000
G = 64
D_IN = 128
D_H = 256
HALF = 128

NC = 2
NS = 16
LANES = 16

CHUNK = 128
IDX_STAGE = 40
E_PAD = 327680
NCHUNK = E_PAD // CHUNK
ROWS_PER_TILE = 624
TAIL_ROWS = N - NS * ROWS_PER_TILE
ACC_ROWS = N + LANES

RB = 2000
NB = N // RB


def _sc_degree_body(edst_hbm, ones_hbm, init_hbm, out_hbm, dst_v, ones_v,
                    deg_sh, sem):
    c = lax.axis_index("c")
    s = lax.axis_index("s")
    rbase = s * ROWS_PER_TILE
    pltpu.sync_copy(init_hbm.at[pl.ds(c * ACC_ROWS + rbase, ROWS_PER_TILE)],
                    deg_sh.at[pl.ds(rbase, ROWS_PER_TILE)])
    @pl.when(s == NS - 1)
    def _():
        pltpu.sync_copy(
            init_hbm.at[pl.ds(c * ACC_ROWS + NS * ROWS_PER_TILE,
                              TAIL_ROWS + LANES)],
            deg_sh.at[pl.ds(NS * ROWS_PER_TILE, TAIL_ROWS + LANES)])
    pltpu.sync_copy(ones_hbm, ones_v)

    per_tile = NCHUNK // (NC * NS)
    cbase = (c * NS + s) * per_tile
    pltpu.sync_copy(edst_hbm.at[pl.ds(cbase, per_tile)], dst_v)
    plsc.subcore_barrier()

    def chunk(k, carry):
        pltpu.async_copy(ones_v, deg_sh.at[dst_v.at[k]], sem, add=True)
        return carry
    lax.fori_loop(0, per_tile, chunk, 0)

    def drain(k, carry):
        pltpu.make_async_copy(ones_v, deg_sh.at[dst_v.at[k]], sem).wait()
        return carry
    lax.fori_loop(0, per_tile, drain, 0)

    plsc.subcore_barrier()
    pltpu.sync_copy(deg_sh.at[pl.ds(rbase, ROWS_PER_TILE)],
                    out_hbm.at[pl.ds(c * N + rbase, ROWS_PER_TILE)])
    @pl.when(s == NS - 1)
    def _():
        pltpu.sync_copy(
            deg_sh.at[pl.ds(NS * ROWS_PER_TILE, TAIL_ROWS)],
            out_hbm.at[pl.ds(c * N + NS * ROWS_PER_TILE, TAIL_ROWS)])


def _sc_degree(edst_p, ones8, deg_init):
    mesh = plsc.VectorSubcoreMesh(core_axis_name="c", subcore_axis_name="s")
    per_tile = NCHUNK // (NC * NS)
    run = pl.kernel(
        _sc_degree_body,
        out_type=jax.ShapeDtypeStruct((2 * N, 8), jnp.float32),
        mesh=mesh,
        scratch_types=[
            pltpu.VMEM((per_tile, CHUNK), jnp.int32),
            pltpu.VMEM((CHUNK, 8), jnp.float32),
            pltpu.VMEM_SHARED((ACC_ROWS, 8), jnp.float32),
            pltpu.SemaphoreType.DMA,
        ],
    )
    return run(edst_p, ones8, deg_init)


def _sc_scatter_body(g_hbm, esrc_hbm, edst_hbm, out_hbm, src_v, dst_v, rows_v,
                     acc_sh, sem, *, edge_split):
    c = lax.axis_index("c")
    s = lax.axis_index("s")
    rbase = s * ROWS_PER_TILE
    pltpu.sync_copy(g_hbm.at[pl.ds(c * N + rbase, ROWS_PER_TILE)],
                    acc_sh.at[pl.ds(rbase, ROWS_PER_TILE)])
    @pl.when(s == NS - 1)
    def _():
        pltpu.sync_copy(
            g_hbm.at[pl.ds(c * N + NS * ROWS_PER_TILE, TAIL_ROWS)],
            acc_sh.at[pl.ds(NS * ROWS_PER_TILE, TAIL_ROWS)])

    rows0, rows1 = rows_v
    gsem0, gsem1 = sem
    if edge_split:
        per_tile = NCHUNK // (NC * NS)
        cbase = (c * NS + s) * per_tile
    else:
        per_tile = NCHUNK // NS
        cbase = s * per_tile
    plsc.subcore_barrier()

    def run_stage(st, carry):
        sbase = cbase + st * IDX_STAGE
        pltpu.sync_copy(esrc_hbm.at[pl.ds(c * NCHUNK + sbase, IDX_STAGE)],
                        src_v)
        pltpu.sync_copy(edst_hbm.at[pl.ds(sbase, IDX_STAGE)], dst_v)

        pltpu.async_copy(g_hbm.at[src_v.at[0]], rows0, gsem0)
        pltpu.async_copy(g_hbm.at[src_v.at[1]], rows1, gsem1)

        def do_chunk(k, rows, gsem, nxt):
            pltpu.make_async_copy(g_hbm.at[src_v.at[k]], rows, gsem).wait()
            pltpu.sync_copy(rows, acc_sh.at[dst_v.at[k]], add=True)
            if nxt:
                pltpu.async_copy(g_hbm.at[src_v.at[k + 2]], rows, gsem)

        def quad(qq, carry2):
            k0 = 4 * qq
            do_chunk(k0, rows0, gsem0, True)
            do_chunk(k0 + 1, rows1, gsem1, True)
            do_chunk(k0 + 2, rows0, gsem0, True)
            do_chunk(k0 + 3, rows1, gsem1, True)
            return carry2
        lax.fori_loop(0, IDX_STAGE // 4 - 1, quad, 0)
        k0 = IDX_STAGE - 4
        do_chunk(k0, rows0, gsem0, True)
        do_chunk(k0 + 1, rows1, gsem1, True)
        do_chunk(k0 + 2, rows0, gsem0, False)
        do_chunk(k0 + 3, rows1, gsem1, False)
        return carry
    lax.fori_loop(0, per_tile // IDX_STAGE, run_stage, 0)

    plsc.subcore_barrier()
    pltpu.sync_copy(acc_sh.at[pl.ds(rbase, ROWS_PER_TILE)],
                    out_hbm.at[pl.ds(c * N + rbase, ROWS_PER_TILE)])
    @pl.when(s == NS - 1)
    def _():
        pltpu.sync_copy(
            acc_sh.at[pl.ds(NS * ROWS_PER_TILE, TAIL_ROWS)],
            out_hbm.at[pl.ds(c * N + NS * ROWS_PER_TILE, TAIL_ROWS)])


def _sc_scatter(g_stack, esrc2, edst_p, width, edge_split=False):
    mesh = plsc.VectorSubcoreMesh(core_axis_name="c", subcore_axis_name="s")
    run = pl.kernel(
        functools.partial(_sc_scatter_body, edge_split=edge_split),
        out_type=jax.ShapeDtypeStruct((2 * N, width), jnp.float32),
        mesh=mesh,
        scratch_types=[
            pltpu.VMEM((IDX_STAGE, CHUNK), jnp.int32),
            pltpu.VMEM((IDX_STAGE, CHUNK), jnp.int32),
            (pltpu.VMEM((CHUNK, width), jnp.float32),
             pltpu.VMEM((CHUNK, width), jnp.float32)),
            pltpu.VMEM_SHARED((ACC_ROWS, width), jnp.float32),
            (pltpu.SemaphoreType.DMA, pltpu.SemaphoreType.DMA),
        ],
    )
    return run(g_stack, esrc2, edst_p)


def _scale1_body(x_ref, dga_ref, dgb_ref, t_ref, dinv_ref):
    deg = dga_ref[:, 0:1] + dgb_ref[:, 0:1]
    dinv = lax.rsqrt(deg)
    t_ref[...] = x_ref[...] * dinv
    dinv_ref[...] = dinv


def _scale1(x, deg_parts):
    return pl.pallas_call(
        _scale1_body,
        grid=(NB, 2),
        in_specs=[
            pl.BlockSpec((RB, D_IN), lambda i, j: (i, 0)),
            pl.BlockSpec((RB, 8), lambda i, j: (i, 0)),
            pl.BlockSpec((RB, 8), lambda i, j: (NB + i, 0)),
        ],
        out_specs=[
            pl.BlockSpec((RB, D_IN), lambda i, j: (j * NB + i, 0)),
            pl.BlockSpec((RB, 1), lambda i, j: (i, 0)),
        ],
        out_shape=[
            jax.ShapeDtypeStruct((2 * N, D_IN), jnp.float32),
            jax.ShapeDtypeStruct((N, 1), jnp.float32),
        ],
        compiler_params=pltpu.CompilerParams(
            dimension_semantics=("arbitrary", "arbitrary")),
    )(x, deg_parts, deg_parts)


def _tca_body(apa_ref, apb_ref, x_ref, dinv_ref, w1_ref, b1_ref, t2_ref):
    dinv = dinv_ref[...]
    acc = apa_ref[...] + apb_ref[...] - x_ref[...] * dinv
    h = jnp.dot(acc, w1_ref[...], preferred_element_type=jnp.float32)
    h = jnp.maximum(h * dinv + b1_ref[...], 0.0)
    t2_ref[...] = h * dinv


def _tca(acc1, x, dinv, W1, b1r):
    return pl.pallas_call(
        _tca_body,
        grid=(NB, 2),
        in_specs=[
            pl.BlockSpec((RB, D_IN), lambda i, j: (i, 0)),
            pl.BlockSpec((RB, D_IN), lambda i, j: (NB + i, 0)),
            pl.BlockSpec((RB, D_IN), lambda i, j: (i, 0)),
            pl.BlockSpec((RB, 1), lambda i, j: (i, 0)),
            pl.BlockSpec((D_IN, HALF), lambda i, j: (0, j)),
            pl.BlockSpec((1, HALF), lambda i, j: (0, j)),
        ],
        out_specs=pl.BlockSpec((RB, HALF), lambda i, j: (j * NB + i, 0)),
        out_shape=jax.ShapeDtypeStruct((2 * N, HALF), jnp.float32),
        compiler_params=pltpu.CompilerParams(
            dimension_semantics=("arbitrary", "arbitrary")),
    )(acc1, acc1, x, dinv, W1, b1r)


def _tcb_body(alo_ref, ahi_ref, dinv_ref, w2_ref, b2_ref, wfc_ref, bfc_ref,
              batch_ref, out_ref, sacc, cacc):
    i = pl.program_id(0)
    acc = jnp.concatenate([alo_ref[...], ahi_ref[...]], axis=1)
    h = jnp.dot(acc, w2_ref[...], preferred_element_type=jnp.float32)
    h = jnp.maximum(h * dinv_ref[...] + b2_ref[...], 0.0)
    y = jnp.dot(h, wfc_ref[...], preferred_element_type=jnp.float32)
    y = y + bfc_ref[0, 0]
    bb = batch_ref[0, 0, :]
    onehot = (bb[:, None] == lax.broadcasted_iota(jnp.int32, (1, G), 1)
              ).astype(jnp.float32)
    ps = jnp.sum(onehot * y, axis=0)[:, None]
    cs = jnp.sum(onehot, axis=0)[:, None]

    @pl.when(i == 0)
    def _():
        sacc[...] = jnp.zeros_like(sacc)
        cacc[...] = jnp.zeros_like(cacc)

    sacc[...] += ps
    cacc[...] += cs

    @pl.when(i == NB - 1)
    def _():
        out_ref[...] = sacc[...] / jnp.maximum(cacc[...], 1.0)


def _tcb(acc2, dinv, W2, b2r, Wfc, bfcr, batch3):
    return pl.pallas_call(
        _tcb_body,
        grid=(NB,),
        in_specs=[
            pl.BlockSpec((RB, HALF), lambda i: (i, 0)),
            pl.BlockSpec((RB, HALF), lambda i: (NB + i, 0)),
            pl.BlockSpec((RB, 1), lambda i: (i, 0)),
            pl.BlockSpec((D_H, D_H), lambda i: (0, 0)),
            pl.BlockSpec((1, D_H), lambda i: (0, 0)),
            pl.BlockSpec((D_H, 1), lambda i: (0, 0)),
            pl.BlockSpec((1, 1), lambda i: (0, 0)),
            pl.BlockSpec((1, 1, RB), lambda i: (i, 0, 0)),
        ],
        out_specs=pl.BlockSpec((G, 1), lambda i: (0, 0)),
        out_shape=jax.ShapeDtypeStruct((G, 1), jnp.float32),
        scratch_shapes=[
            pltpu.VMEM((G, 1), jnp.float32),
            pltpu.VMEM((G, 1), jnp.float32),
        ],
        compiler_params=pltpu.CompilerParams(
            dimension_semantics=("arbitrary",)),
    )(acc2, acc2, dinv, W2, b2r, Wfc, bfcr, batch3)


def kernel(x, edge_index, batch, W1, b1, W2, b2, Wfc, bfc):
    esrc = edge_index[0]
    edst = edge_index[1]
    pad = E_PAD - E
    pad_src = (jnp.arange(pad, dtype=jnp.int32) * 131) % N
    pad_dst = N + (jnp.arange(pad, dtype=jnp.int32) % LANES)
    esrc_p = jnp.concatenate([esrc, pad_src]).reshape(NCHUNK, CHUNK)
    edst_p = jnp.concatenate([edst, pad_dst]).reshape(NCHUNK, CHUNK)
    esrc2 = jnp.concatenate([esrc_p, esrc_p + N])

    ones8 = jnp.ones((CHUNK, 8), jnp.float32)
    deg_init = jnp.concatenate([
        jnp.ones((ACC_ROWS, 8), jnp.float32),
        jnp.zeros((ACC_ROWS, 8), jnp.float32)])

    b1r = b1.reshape(1, D_H)
    b2r = b2.reshape(1, D_H)
    bfcr = bfc.reshape(1, 1)
    batch3 = batch.reshape(NB, 1, RB)

    deg_parts = _sc_degree(edst_p, ones8, deg_init)
    t1d, dinv = _scale1(x, deg_parts)
    acc1 = _sc_scatter(t1d, esrc2, edst_p, D_IN, edge_split=True)
    t2 = _tca(acc1, x, dinv, W1, b1r)
    acc2 = _sc_scatter(t2, esrc2, edst_p, HALF)
    return _tcb(acc2, dinv, W2, b2r, Wfc, bfcr, batch3)
